# baseline (speedup 1.0000x reference)
import functools
import math

import jax
import jax.numpy as jnp
from jax import lax
from jax.experimental import pallas as pl
from jax.experimental.pallas import tpu as pltpu
from jax.experimental.pallas import tpu_sc as plsc

_NW = 32
_NBUF = 4
_L = 16


def _emb_call(B0, B1, V, D):
    t_per_w = B0 // _NW
    mesh = plsc.VectorSubcoreMesh(core_axis_name="c", subcore_axis_name="s")

    @functools.partial(
        pl.kernel,
        mesh=mesh,
        compiler_params=pltpu.CompilerParams(needs_layout_passes=False),
        out_type=jax.ShapeDtypeStruct((B1, D, B0), jnp.float32),
        scratch_types=[
            pltpu.VMEM((B1, t_per_w), jnp.int32),
            pltpu.VMEM((_NBUF, t_per_w, 2 * D), jnp.float32),
            pltpu.VMEM((_NBUF, D, t_per_w), jnp.float32),
            pltpu.SemaphoreType.DMA((_NBUF,)),
            pltpu.SemaphoreType.DMA((_NBUF,)),
        ],
    )
    def emb_kernel(xt_hbm, lut_hbm, out_hbm, idx_v, rows_v, tr_v, in_sem, out_sem):
        scale = jnp.float32(math.sqrt(D))
        wid = lax.axis_index("s") * 2 + lax.axis_index("c")
        t0 = wid * t_per_w
        pltpu.sync_copy(xt_hbm.at[:, pl.ds(t0, t_per_w)], idx_v)

        def gather(g, b):
            pltpu.async_copy(lut_hbm.at[idx_v.at[g]], rows_v.at[b], in_sem.at[b])

        def wait_in(b):
            pltpu.make_async_copy(
                lut_hbm.at[idx_v.at[0]], rows_v.at[b], in_sem.at[b]
            ).wait()

        def put(g, b):
            pltpu.async_copy(
                tr_v.at[b], out_hbm.at[g, :, pl.ds(t0, t_per_w)], out_sem.at[b]
            )

        def wait_out(b):
            pltpu.make_async_copy(
                tr_v.at[b], out_hbm.at[0, :, pl.ds(t0, t_per_w)], out_sem.at[b]
            ).wait()

        for b in range(_NBUF):
            gather(b, b)

        iota = lax.iota(jnp.int32, _L)
        rot = [((iota + j) & (_L - 1)) for j in range(_L)]

        def step(g, b):
            wait_in(b)

            @pl.when(g >= _NBUF)
            def _():
                wait_out(b)

            rows2d = rows_v.at[b]
            tr2d = tr_v.at[b]

            for dd in range(D // _L):
                cols = [rot[j] + (dd * _L) for j in range(_L)]

                def blk(k, _, cols=cols):
                    r = iota + k * _L
                    for j in range(_L):
                        v = plsc.load_gather(rows2d, [r, cols[j]])
                        plsc.store_scatter(tr2d, [cols[j], r], v * scale)
                    return 0

                lax.fori_loop(0, t_per_w // _L, blk, 0)

            @pl.when(g + _NBUF < B1)
            def _():
                gather(g + _NBUF, b)

            put(g, b)

        def outer(i, _):
            g0 = i * _NBUF
            for b in range(_NBUF):
                step(g0 + b, b)
            return 0

        n_full = B1 // _NBUF
        lax.fori_loop(0, n_full, outer, 0)
        for t in range(B1 - n_full * _NBUF):
            step(n_full * _NBUF + t, t)
        for b in range(_NBUF):
            wait_out(b)

    return emb_kernel


def kernel(x, lut):
    B0, B1 = x.shape
    V, D = lut.shape
    lut_pad = jnp.pad(lut, ((0, 0), (0, D)))
    xt = x.T.astype(jnp.int32)
    out_t = _emb_call(B0, B1, V, D)(xt, lut_pad)
    return jnp.transpose(out_t, (2, 0, 1))


# speedup vs baseline: 1.1319x; 1.1319x over previous
import functools
import math

import jax
import jax.numpy as jnp
from jax import lax
from jax.experimental import pallas as pl
from jax.experimental.pallas import tpu as pltpu
from jax.experimental.pallas import tpu_sc as plsc

_NW = 32
_NBUF = 5
_L = 16


def _emb_call(B0, B1, V, D):
    t_per_w = B0 // _NW
    mesh = plsc.VectorSubcoreMesh(core_axis_name="c", subcore_axis_name="s")

    @functools.partial(
        pl.kernel,
        mesh=mesh,
        compiler_params=pltpu.CompilerParams(needs_layout_passes=False),
        out_type=jax.ShapeDtypeStruct((B1, D, B0), jnp.float32),
        scratch_types=[
            pltpu.VMEM((B1, t_per_w), jnp.int32),
            pltpu.VMEM((_NBUF, t_per_w, 2 * D), jnp.float32),
            pltpu.VMEM((_NBUF, D, t_per_w), jnp.float32),
            pltpu.SemaphoreType.DMA((_NBUF,)),
            pltpu.SemaphoreType.DMA((_NBUF,)),
        ],
    )
    def emb_kernel(xt_hbm, lut_hbm, out_hbm, idx_v, rows_v, tr_v, in_sem, out_sem):
        scale = jnp.float32(math.sqrt(D))
        wid = lax.axis_index("s") * 2 + lax.axis_index("c")
        t0 = wid * t_per_w
        pltpu.sync_copy(xt_hbm.at[:, pl.ds(t0, t_per_w)], idx_v)

        def gather(g, b):
            pltpu.async_copy(lut_hbm.at[idx_v.at[g]], rows_v.at[b], in_sem.at[b])

        def wait_in(b):
            pltpu.make_async_copy(
                lut_hbm.at[idx_v.at[0]], rows_v.at[b], in_sem.at[b]
            ).wait()

        def put(g, b):
            pltpu.async_copy(
                tr_v.at[b], out_hbm.at[g, :, pl.ds(t0, t_per_w)], out_sem.at[b]
            )

        def wait_out(b):
            pltpu.make_async_copy(
                tr_v.at[b], out_hbm.at[0, :, pl.ds(t0, t_per_w)], out_sem.at[b]
            ).wait()

        for b in range(_NBUF):
            gather(b, b)

        iota = lax.iota(jnp.int32, _L)
        rot = [((iota + j) & (_L - 1)) for j in range(_L)]

        def step(g, b):
            wait_in(b)

            @pl.when(g >= _NBUF)
            def _():
                wait_out(b)

            rows2d = rows_v.at[b]
            tr2d = tr_v.at[b]

            def blk(k, _):
                tt0 = (k // (D // _L)) * _L
                dd0 = (k % (D // _L)) * _L
                r = iota + tt0
                for j in range(_L):
                    c = rot[j] + dd0
                    v = plsc.load_gather(rows2d, [r, c])
                    plsc.store_scatter(tr2d, [c, r], v * scale)
                return 0

            lax.fori_loop(0, (t_per_w // _L) * (D // _L), blk, 0)

            @pl.when(g + _NBUF < B1)
            def _():
                gather(g + _NBUF, b)

            put(g, b)

        def outer(i, _):
            g0 = i * _NBUF
            for b in range(_NBUF):
                step(g0 + b, b)
            return 0

        n_full = B1 // _NBUF
        lax.fori_loop(0, n_full, outer, 0)
        for t in range(B1 - n_full * _NBUF):
            step(n_full * _NBUF + t, t)
        for b in range(_NBUF):
            wait_out(b)

    return emb_kernel


def kernel(x, lut):
    B0, B1 = x.shape
    V, D = lut.shape
    lut_pad = jnp.pad(lut, ((0, 0), (0, D)))
    xt = x.T.astype(jnp.int32)
    out_t = _emb_call(B0, B1, V, D)(xt, lut_pad)
    return jnp.transpose(out_t, (2, 0, 1))


# speedup vs baseline: 1.1356x; 1.0033x over previous
import functools
import math

import jax
import jax.numpy as jnp
from jax import lax
from jax.experimental import pallas as pl
from jax.experimental.pallas import tpu as pltpu
from jax.experimental.pallas import tpu_sc as plsc

_NW = 32
_NBUF = 4
_L = 16


def _emb_call(B0, B1, V, D):
    t_per_w = B0 // _NW
    mesh = plsc.VectorSubcoreMesh(core_axis_name="c", subcore_axis_name="s")

    @functools.partial(
        pl.kernel,
        mesh=mesh,
        compiler_params=pltpu.CompilerParams(needs_layout_passes=False),
        out_type=jax.ShapeDtypeStruct((B1, D, B0), jnp.float32),
        scratch_types=[
            pltpu.VMEM((B1, t_per_w), jnp.int32),
            pltpu.VMEM((_NBUF, t_per_w, 2 * D), jnp.float32),
            pltpu.VMEM((_NBUF, D, t_per_w), jnp.float32),
            pltpu.SemaphoreType.DMA((_NBUF,)),
            pltpu.SemaphoreType.DMA((_NBUF,)),
        ],
    )
    def emb_kernel(xt_hbm, lut_hbm, out_hbm, idx_v, rows_v, tr_v, in_sem, out_sem):
        scale = jnp.float32(math.sqrt(D))
        wid = lax.axis_index("s") * 2 + lax.axis_index("c")
        t0 = wid * t_per_w
        pltpu.sync_copy(xt_hbm.at[:, pl.ds(t0, t_per_w)], idx_v)

        def gather(g, b):
            pltpu.async_copy(lut_hbm.at[idx_v.at[g]], rows_v.at[b], in_sem.at[b])

        def wait_in(b):
            pltpu.make_async_copy(
                lut_hbm.at[idx_v.at[0]], rows_v.at[b], in_sem.at[b]
            ).wait()

        def put(g, b):
            pltpu.async_copy(
                tr_v.at[b], out_hbm.at[g, :, pl.ds(t0, t_per_w)], out_sem.at[b]
            )

        def wait_out(b):
            pltpu.make_async_copy(
                tr_v.at[b], out_hbm.at[0, :, pl.ds(t0, t_per_w)], out_sem.at[b]
            ).wait()

        for b in range(_NBUF):
            gather(b, b)

        iota = lax.iota(jnp.int32, _L)
        rot = [((iota + j) & (_L - 1)) for j in range(_L)]

        def step(g, b):
            wait_in(b)

            @pl.when(g >= _NBUF)
            def _():
                wait_out(b)

            rows2d = rows_v.at[b]
            tr2d = tr_v.at[b]

            def blk(k, _):
                tt0 = (k // (D // _L)) * _L
                dd0 = (k % (D // _L)) * _L
                r = iota + tt0
                for j in range(_L):
                    c = rot[j] + dd0
                    v = plsc.load_gather(rows2d, [r, c])
                    plsc.store_scatter(tr2d, [c, r], v * scale)
                return 0

            lax.fori_loop(0, (t_per_w // _L) * (D // _L), blk, 0)

            @pl.when(g + _NBUF < B1)
            def _():
                gather(g + _NBUF, b)

            put(g, b)

        def outer(i, _):
            g0 = i * _NBUF
            for b in range(_NBUF):
                step(g0 + b, b)
            return 0

        n_full = B1 // _NBUF
        lax.fori_loop(0, n_full, outer, 0)
        for t in range(B1 - n_full * _NBUF):
            step(n_full * _NBUF + t, t)
        for b in range(_NBUF):
            wait_out(b)

    return emb_kernel


def kernel(x, lut):
    B0, B1 = x.shape
    V, D = lut.shape
    lut_pad = jnp.pad(lut, ((0, 0), (0, D)))
    xt = x.T.astype(jnp.int32)
    out_t = _emb_call(B0, B1, V, D)(xt, lut_pad)
    return jnp.transpose(out_t, (2, 0, 1))
